# baseline (speedup 1.0000x reference)
import functools

import jax
import jax.numpy as jnp
from jax import lax
from jax.experimental import pallas as pl
from jax.experimental.pallas import tpu as pltpu
from jax.experimental.pallas import tpu_sc as plsc

B, S, D, V = 1024, 200, 128, 100000
NC, NS = 2, 16
NW = NC * NS
BW = B // NW
PQ = 2
C = PQ * BW
CHUNKS = S // PQ
LANES = 16
CD = D // LANES
NBUF = 4


def _matryoshka_sc(src4, oidx4, E0, E1, E2, P0f, P1f, P2f):
    mesh = plsc.VectorSubcoreMesh(core_axis_name="c", subcore_axis_name="s")

    @functools.partial(
        pl.kernel,
        mesh=mesh,
        out_type=jax.ShapeDtypeStruct((B * S, D), jnp.float32),
        scratch_types=[
            pltpu.VMEM((CHUNKS, C), jnp.int32),
            pltpu.VMEM((CHUNKS, C), jnp.int32),
            pltpu.VMEM((NBUF, C, D), jnp.float32),
            pltpu.VMEM((NBUF, C, D), jnp.float32),
            pltpu.VMEM((NBUF, C, D), jnp.float32),
            pltpu.VMEM((NBUF, 3, PQ, D), jnp.float32),
            pltpu.SemaphoreType.DMA,
            pltpu.SemaphoreType.DMA,
            pltpu.SemaphoreType.DMA,
            pltpu.SemaphoreType.DMA,
            pltpu.SemaphoreType.DMA,
            pltpu.SemaphoreType.DMA,
            pltpu.SemaphoreType.DMA,
            pltpu.SemaphoreType.DMA,
        ],
    )
    def k(src_hbm, oidx_hbm, e0, e1, e2, p0, p1, p2, out_hbm,
          idx_all, oidx_all, acc, g1, g2, pbuf,
          sg0, sg1, sg2, sg3, so0, so1, so2, so3):
        semg = (sg0, sg1, sg2, sg3)
        semo = (so0, so1, so2, so3)
        wid = lax.axis_index("s") * NC + lax.axis_index("c")

        pltpu.sync_copy(src_hbm.at[wid], idx_all)
        pltpu.sync_copy(oidx_hbm.at[wid], oidx_all)

        def issue_far(ck, b):
            idx = idx_all.at[ck]
            pltpu.async_copy(e1.at[idx], g1.at[b], semg[b])
            pltpu.async_copy(e2.at[idx], g2.at[b], semg[b])
            for t, ptab in enumerate((p0, p1, p2)):
                pltpu.async_copy(ptab.at[pl.ds(ck * PQ, PQ)],
                                 pbuf.at[b, t], semg[b])

        def issue_e0(ck, b):
            pltpu.async_copy(e0.at[idx_all.at[ck]], acc.at[b], semg[b])

        def wait_chunk(b):
            pltpu.make_async_copy(e0.at[pl.ds(0, C)], acc.at[b],
                                  semg[b]).wait()
            pltpu.make_async_copy(e0.at[pl.ds(0, C)], g1.at[b],
                                  semg[b]).wait()
            pltpu.make_async_copy(e0.at[pl.ds(0, C)], g2.at[b],
                                  semg[b]).wait()
            for t in range(3):
                pltpu.make_async_copy(p0.at[pl.ds(0, PQ)], pbuf.at[b, t],
                                      semg[b]).wait()

        def wait_store(b):
            pltpu.make_async_copy(acc.at[b], out_hbm.at[oidx_all.at[0]],
                                  semo[b]).wait()

        for ck in range(NBUF):
            issue_far(ck, ck)
        for ck in range(2):
            issue_e0(ck, ck)

        def step(ck, b):
            eb = (b + 2) % NBUF

            @pl.when(ck >= 2)
            def _():
                wait_store(eb)

            @pl.when(ck + 2 < CHUNKS)
            def _():
                issue_e0(ck + 2, eb)

            wait_chunk(b)

            for q in range(PQ):
                pos = []
                for c in range(CD):
                    sl = pl.ds(c * LANES, LANES)
                    pos.append(pbuf[b, 0, q, sl] + pbuf[b, 1, q, sl]
                               + pbuf[b, 2, q, sl])

                def row_body(r, _pos=pos):
                    for c in range(CD):
                        sl = pl.ds(c * LANES, LANES)
                        plsc.addupdate(acc.at[b, r, sl],
                                       g1[b, r, sl] + g2[b, r, sl] + _pos[c])

                plsc.parallel_loop(q * BW, (q + 1) * BW, 1,
                                   unroll=8)(row_body)

            pltpu.async_copy(acc.at[b], out_hbm.at[oidx_all.at[ck]], semo[b])

            @pl.when(ck + NBUF < CHUNKS)
            def _():
                issue_far(ck + NBUF, b)

        def body(i, carry):
            for b in range(NBUF):
                step(i * NBUF + b, b)
            return carry

        lax.fori_loop(0, CHUNKS // NBUF, body, 0)
        wait_store(2)
        wait_store(3)

    return k(src4, oidx4, E0, E1, E2, P0f, P1f, P2f)


def kernel(src, E0, E1, E2, P0, P1, P2):
    src4 = src.reshape(NW, BW, CHUNKS, PQ).transpose(0, 2, 3, 1)
    src4 = src4.reshape(NW, CHUNKS, C)
    brow = (jnp.arange(NW)[:, None, None, None] * BW
            + jnp.arange(BW)[None, None, None, :])
    spos = (jnp.arange(CHUNKS)[None, :, None, None] * PQ
            + jnp.arange(PQ)[None, None, :, None])
    oidx4 = (brow * S + spos).astype(jnp.int32).reshape(NW, CHUNKS, C)
    P0f = P0.reshape(-1, D)
    P1f = P1.reshape(-1, D)
    P2f = P2.reshape(-1, D)
    out = _matryoshka_sc(src4, oidx4, E0, E1, E2, P0f, P1f, P2f)
    return out.reshape(B, S, D)


# speedup vs baseline: 1.1619x; 1.1619x over previous
import functools

import jax
import jax.numpy as jnp
from jax import lax
from jax.experimental import pallas as pl
from jax.experimental.pallas import tpu as pltpu
from jax.experimental.pallas import tpu_sc as plsc

B, S, D, V = 1024, 200, 128, 100000
NC, NS = 2, 16
NW = NC * NS
BW = B // NW
PQ = 2
C = PQ * BW
CHUNKS = S // PQ
LANES = 16
CD = D // LANES
NBUF = 4


def _matryoshka_sc(src4, oidx4, E0, E1, E2, P0f, P1f, P2f):
    mesh = plsc.VectorSubcoreMesh(core_axis_name="c", subcore_axis_name="s")

    @functools.partial(
        pl.kernel,
        mesh=mesh,
        out_type=jax.ShapeDtypeStruct((B * S, D), jnp.float32),
        scratch_types=[
            pltpu.VMEM((CHUNKS, C), jnp.int32),
            pltpu.VMEM((CHUNKS, C), jnp.int32),
            pltpu.VMEM((NBUF, C, D), jnp.float32),
            pltpu.VMEM((NBUF, 3, PQ, D), jnp.float32),
            pltpu.SemaphoreType.DMA,
            pltpu.SemaphoreType.DMA,
            pltpu.SemaphoreType.DMA,
            pltpu.SemaphoreType.DMA,
            pltpu.SemaphoreType.DMA,
            pltpu.SemaphoreType.DMA,
            pltpu.SemaphoreType.DMA,
            pltpu.SemaphoreType.DMA,
            pltpu.SemaphoreType.DMA,
            pltpu.SemaphoreType.DMA,
            pltpu.SemaphoreType.DMA,
            pltpu.SemaphoreType.DMA,
        ],
    )
    def k(src_hbm, oidx_hbm, e0, e1, e2, p0, p1, p2, out_hbm,
          idx_all, oidx_all, acc, pbuf,
          sg0, sg1, sg2, sg3, sa0, sa1, sa2, sa3, so0, so1, so2, so3):
        semg = (sg0, sg1, sg2, sg3)
        sema = (sa0, sa1, sa2, sa3)
        semo = (so0, so1, so2, so3)
        wid = lax.axis_index("s") * NC + lax.axis_index("c")

        pltpu.sync_copy(src_hbm.at[wid], idx_all)
        pltpu.sync_copy(oidx_hbm.at[wid], oidx_all)

        def issue_e0(ck, b):
            pltpu.async_copy(e0.at[idx_all.at[ck]], acc.at[b], semg[b])

        def issue_adds(ck, b):
            idx = idx_all.at[ck]
            pltpu.async_copy(e1.at[idx], acc.at[b], sema[b], add=True)
            pltpu.async_copy(e2.at[idx], acc.at[b], sema[b], add=True)
            for t, ptab in enumerate((p0, p1, p2)):
                pltpu.async_copy(ptab.at[pl.ds(ck * PQ, PQ)],
                                 pbuf.at[b, t], sema[b])

        def wait_e0(b):
            pltpu.make_async_copy(e0.at[pl.ds(0, C)], acc.at[b],
                                  semg[b]).wait()

        def wait_adds(b):
            pltpu.make_async_copy(e0.at[pl.ds(0, C)], acc.at[b],
                                  sema[b]).wait()
            pltpu.make_async_copy(e0.at[pl.ds(0, C)], acc.at[b],
                                  sema[b]).wait()
            for t in range(3):
                pltpu.make_async_copy(p0.at[pl.ds(0, PQ)], pbuf.at[b, t],
                                      sema[b]).wait()

        def wait_store(b):
            pltpu.make_async_copy(acc.at[b], out_hbm.at[oidx_all.at[0]],
                                  semo[b]).wait()

        issue_e0(0, 0)
        issue_e0(1, 1)
        wait_e0(0)
        issue_adds(0, 0)

        def step(ck, b):
            eb2 = (b + 2) % NBUF
            eb1 = (b + 1) % NBUF

            @pl.when(ck >= 2)
            def _():
                wait_store(eb2)

            @pl.when(ck + 2 < CHUNKS)
            def _():
                issue_e0(ck + 2, eb2)

            @pl.when(ck + 1 < CHUNKS)
            def _():
                wait_e0(eb1)
                issue_adds(ck + 1, eb1)

            wait_adds(b)

            for q in range(PQ):
                pos = []
                for c in range(CD):
                    sl = pl.ds(c * LANES, LANES)
                    pos.append(pbuf[b, 0, q, sl] + pbuf[b, 1, q, sl]
                               + pbuf[b, 2, q, sl])

                def row_body(r, _pos=pos):
                    for c in range(CD):
                        sl = pl.ds(c * LANES, LANES)
                        plsc.addupdate(acc.at[b, r, sl], _pos[c])

                plsc.parallel_loop(q * BW, (q + 1) * BW, 1,
                                   unroll=4)(row_body)

            pltpu.async_copy(acc.at[b], out_hbm.at[oidx_all.at[ck]], semo[b])

        def body(i, carry):
            for b in range(NBUF):
                step(i * NBUF + b, b)
            return carry

        lax.fori_loop(0, CHUNKS // NBUF, body, 0)
        wait_store(2)
        wait_store(3)

    return k(src4, oidx4, E0, E1, E2, P0f, P1f, P2f)


def kernel(src, E0, E1, E2, P0, P1, P2):
    src4 = src.reshape(NW, BW, CHUNKS, PQ).transpose(0, 2, 3, 1)
    src4 = src4.reshape(NW, CHUNKS, C)
    brow = (jnp.arange(NW)[:, None, None, None] * BW
            + jnp.arange(BW)[None, None, None, :])
    spos = (jnp.arange(CHUNKS)[None, :, None, None] * PQ
            + jnp.arange(PQ)[None, None, :, None])
    oidx4 = (brow * S + spos).astype(jnp.int32).reshape(NW, CHUNKS, C)
    P0f = P0.reshape(-1, D)
    P1f = P1.reshape(-1, D)
    P2f = P2.reshape(-1, D)
    out = _matryoshka_sc(src4, oidx4, E0, E1, E2, P0f, P1f, P2f)
    return out.reshape(B, S, D)


# speedup vs baseline: 1.1967x; 1.0300x over previous
import functools

import jax
import jax.numpy as jnp
from jax import lax
from jax.experimental import pallas as pl
from jax.experimental.pallas import tpu as pltpu
from jax.experimental.pallas import tpu_sc as plsc

B, S, D, V = 1024, 200, 128, 100000
NC, NS = 2, 16
NW = NC * NS
BW = B // NW
PQ = 4
C = PQ * BW
CHUNKS = S // PQ
LANES = 16
CD = D // LANES
NBUF = 4


def _matryoshka_sc(src4, oidx4, E0, E1, E2, P0f, P1f, P2f):
    mesh = plsc.VectorSubcoreMesh(core_axis_name="c", subcore_axis_name="s")

    @functools.partial(
        pl.kernel,
        mesh=mesh,
        out_type=jax.ShapeDtypeStruct((B * S, D), jnp.float32),
        scratch_types=[
            pltpu.VMEM((CHUNKS, C), jnp.int32),
            pltpu.VMEM((CHUNKS, C), jnp.int32),
            pltpu.VMEM((NBUF, C, D), jnp.float32),
            pltpu.VMEM((NBUF, 3, PQ, D), jnp.float32),
            pltpu.SemaphoreType.DMA,
            pltpu.SemaphoreType.DMA,
            pltpu.SemaphoreType.DMA,
            pltpu.SemaphoreType.DMA,
            pltpu.SemaphoreType.DMA,
            pltpu.SemaphoreType.DMA,
            pltpu.SemaphoreType.DMA,
            pltpu.SemaphoreType.DMA,
            pltpu.SemaphoreType.DMA,
            pltpu.SemaphoreType.DMA,
            pltpu.SemaphoreType.DMA,
            pltpu.SemaphoreType.DMA,
        ],
    )
    def k(src_hbm, oidx_hbm, e0, e1, e2, p0, p1, p2, out_hbm,
          idx_all, oidx_all, acc, pbuf,
          sg0, sg1, sg2, sg3, sa0, sa1, sa2, sa3, so0, so1, so2, so3):
        semg = (sg0, sg1, sg2, sg3)
        sema = (sa0, sa1, sa2, sa3)
        semo = (so0, so1, so2, so3)
        wid = lax.axis_index("s") * NC + lax.axis_index("c")

        pltpu.sync_copy(src_hbm.at[wid], idx_all)
        pltpu.sync_copy(oidx_hbm.at[wid], oidx_all)

        def issue_e0(ck, b):
            pltpu.async_copy(e0.at[idx_all.at[ck]], acc.at[b], semg[b])

        def issue_adds(ck, b):
            idx = idx_all.at[ck]
            pltpu.async_copy(e1.at[idx], acc.at[b], sema[b], add=True)
            pltpu.async_copy(e2.at[idx], acc.at[b], sema[b], add=True)
            for t, ptab in enumerate((p0, p1, p2)):
                pltpu.async_copy(ptab.at[pl.ds(ck * PQ, PQ)],
                                 pbuf.at[b, t], sema[b])

        def wait_e0(b):
            pltpu.make_async_copy(e0.at[pl.ds(0, C)], acc.at[b],
                                  semg[b]).wait()

        def wait_adds(b):
            pltpu.make_async_copy(e0.at[pl.ds(0, C)], acc.at[b],
                                  sema[b]).wait()
            pltpu.make_async_copy(e0.at[pl.ds(0, C)], acc.at[b],
                                  sema[b]).wait()
            for t in range(3):
                pltpu.make_async_copy(p0.at[pl.ds(0, PQ)], pbuf.at[b, t],
                                      sema[b]).wait()

        def wait_store(b):
            pltpu.make_async_copy(acc.at[b], out_hbm.at[oidx_all.at[0]],
                                  semo[b]).wait()

        issue_e0(0, 0)
        issue_e0(1, 1)
        wait_e0(0)
        issue_adds(0, 0)

        def step(ck, b):
            eb2 = (b + 2) % NBUF
            eb1 = (b + 1) % NBUF

            @pl.when(ck >= 2)
            def _():
                wait_store(eb2)

            @pl.when(ck + 2 < CHUNKS)
            def _():
                issue_e0(ck + 2, eb2)

            @pl.when(ck + 1 < CHUNKS)
            def _():
                wait_e0(eb1)
                issue_adds(ck + 1, eb1)

            wait_adds(b)

            for q in range(PQ):
                pos = []
                for c in range(CD):
                    sl = pl.ds(c * LANES, LANES)
                    pos.append(pbuf[b, 0, q, sl] + pbuf[b, 1, q, sl]
                               + pbuf[b, 2, q, sl])

                def row_body(r, _pos=pos):
                    for c in range(CD):
                        sl = pl.ds(c * LANES, LANES)
                        plsc.addupdate(acc.at[b, r, sl], _pos[c])

                plsc.parallel_loop(q * BW, (q + 1) * BW, 1,
                                   unroll=4)(row_body)

            pltpu.async_copy(acc.at[b], out_hbm.at[oidx_all.at[ck]], semo[b])

        def body(i, carry):
            for b in range(NBUF):
                step(i * NBUF + b, b)
            return carry

        lax.fori_loop(0, CHUNKS // NBUF, body, 0)
        for tck in range(CHUNKS - CHUNKS % NBUF, CHUNKS):
            step(jnp.int32(tck), tck % NBUF)
        wait_store((CHUNKS - 2) % NBUF)
        wait_store((CHUNKS - 1) % NBUF)

    return k(src4, oidx4, E0, E1, E2, P0f, P1f, P2f)


def kernel(src, E0, E1, E2, P0, P1, P2):
    src4 = src.reshape(NW, BW, CHUNKS, PQ).transpose(0, 2, 3, 1)
    src4 = src4.reshape(NW, CHUNKS, C)
    brow = (jnp.arange(NW)[:, None, None, None] * BW
            + jnp.arange(BW)[None, None, None, :])
    spos = (jnp.arange(CHUNKS)[None, :, None, None] * PQ
            + jnp.arange(PQ)[None, None, :, None])
    oidx4 = (brow * S + spos).astype(jnp.int32).reshape(NW, CHUNKS, C)
    P0f = P0.reshape(-1, D)
    P1f = P1.reshape(-1, D)
    P2f = P2.reshape(-1, D)
    out = _matryoshka_sc(src4, oidx4, E0, E1, E2, P0f, P1f, P2f)
    return out.reshape(B, S, D)


# speedup vs baseline: 1.2030x; 1.0053x over previous
import functools

import jax
import jax.numpy as jnp
from jax import lax
from jax.experimental import pallas as pl
from jax.experimental.pallas import tpu as pltpu
from jax.experimental.pallas import tpu_sc as plsc

B, S, D, V = 1024, 200, 128, 100000
NC, NS = 2, 16
NW = NC * NS
BW = B // NW
PQ = 4
C = PQ * BW
CHUNKS = S // PQ
LANES = 16
CD = D // LANES
NBUF = 6


def _matryoshka_sc(src4, oidx4, E0, E1, E2, P0f, P1f, P2f):
    mesh = plsc.VectorSubcoreMesh(core_axis_name="c", subcore_axis_name="s")

    @functools.partial(
        pl.kernel,
        mesh=mesh,
        out_type=jax.ShapeDtypeStruct((B * S, D), jnp.float32),
        scratch_types=[
            pltpu.VMEM((CHUNKS, C), jnp.int32),
            pltpu.VMEM((CHUNKS, C), jnp.int32),
            pltpu.VMEM((NBUF, C, D), jnp.float32),
            pltpu.VMEM((NBUF, 3, PQ, D), jnp.float32),
        ] + [pltpu.SemaphoreType.DMA] * (3 * NBUF) + [
        ],
    )
    def k(src_hbm, oidx_hbm, e0, e1, e2, p0, p1, p2, out_hbm,
          idx_all, oidx_all, acc, pbuf, *sems):
        semg = sems[:NBUF]
        sema = sems[NBUF:2 * NBUF]
        semo = sems[2 * NBUF:]
        wid = lax.axis_index("s") * NC + lax.axis_index("c")

        pltpu.sync_copy(src_hbm.at[wid], idx_all)
        pltpu.sync_copy(oidx_hbm.at[wid], oidx_all)

        def issue_e0(ck, b):
            pltpu.async_copy(e0.at[idx_all.at[ck]], acc.at[b], semg[b])

        def issue_adds(ck, b):
            idx = idx_all.at[ck]
            pltpu.async_copy(e1.at[idx], acc.at[b], sema[b], add=True)
            pltpu.async_copy(e2.at[idx], acc.at[b], sema[b], add=True)
            for t, ptab in enumerate((p0, p1, p2)):
                pltpu.async_copy(ptab.at[pl.ds(ck * PQ, PQ)],
                                 pbuf.at[b, t], sema[b])

        def wait_e0(b):
            pltpu.make_async_copy(e0.at[pl.ds(0, C)], acc.at[b],
                                  semg[b]).wait()

        def wait_adds(b):
            pltpu.make_async_copy(e0.at[pl.ds(0, C)], acc.at[b],
                                  sema[b]).wait()
            pltpu.make_async_copy(e0.at[pl.ds(0, C)], acc.at[b],
                                  sema[b]).wait()
            for t in range(3):
                pltpu.make_async_copy(p0.at[pl.ds(0, PQ)], pbuf.at[b, t],
                                      sema[b]).wait()

        def wait_store(b):
            pltpu.make_async_copy(acc.at[b], out_hbm.at[oidx_all.at[0]],
                                  semo[b]).wait()

        issue_e0(0, 0)
        issue_e0(1, 1)
        wait_e0(0)
        issue_adds(0, 0)

        def step(ck, b):
            eb2 = (b + 2) % NBUF
            eb1 = (b + 1) % NBUF

            @pl.when(ck >= NBUF - 2)
            def _():
                wait_store(eb2)

            @pl.when(ck + 2 < CHUNKS)
            def _():
                issue_e0(ck + 2, eb2)

            @pl.when(ck + 1 < CHUNKS)
            def _():
                wait_e0(eb1)
                issue_adds(ck + 1, eb1)

            wait_adds(b)

            for q in range(PQ):
                pos = []
                for c in range(CD):
                    sl = pl.ds(c * LANES, LANES)
                    pos.append(pbuf[b, 0, q, sl] + pbuf[b, 1, q, sl]
                               + pbuf[b, 2, q, sl])

                def row_body(r, _pos=pos):
                    for c in range(CD):
                        sl = pl.ds(c * LANES, LANES)
                        plsc.addupdate(acc.at[b, r, sl], _pos[c])

                plsc.parallel_loop(q * BW, (q + 1) * BW, 1,
                                   unroll=4)(row_body)

            pltpu.async_copy(acc.at[b], out_hbm.at[oidx_all.at[ck]], semo[b])

        def body(i, carry):
            for b in range(NBUF):
                step(i * NBUF + b, b)
            return carry

        lax.fori_loop(0, CHUNKS // NBUF, body, 0)
        for tck in range(CHUNKS - CHUNKS % NBUF, CHUNKS):
            step(jnp.int32(tck), tck % NBUF)
        for tck in range(CHUNKS - (NBUF - 2), CHUNKS):
            wait_store(tck % NBUF)

    return k(src4, oidx4, E0, E1, E2, P0f, P1f, P2f)


def kernel(src, E0, E1, E2, P0, P1, P2):
    src4 = src.reshape(NW, BW, CHUNKS, PQ).transpose(0, 2, 3, 1)
    src4 = src4.reshape(NW, CHUNKS, C)
    brow = (jnp.arange(NW)[:, None, None, None] * BW
            + jnp.arange(BW)[None, None, None, :])
    spos = (jnp.arange(CHUNKS)[None, :, None, None] * PQ
            + jnp.arange(PQ)[None, None, :, None])
    oidx4 = (brow * S + spos).astype(jnp.int32).reshape(NW, CHUNKS, C)
    P0f = P0.reshape(-1, D)
    P1f = P1.reshape(-1, D)
    P2f = P2.reshape(-1, D)
    out = _matryoshka_sc(src4, oidx4, E0, E1, E2, P0f, P1f, P2f)
    return out.reshape(B, S, D)
